# Initial kernel scaffold; baseline (speedup 1.0000x reference)
import jax
import jax.numpy as jnp
from jax.experimental import pallas as pl


def kernel(params, edge_index):
    raise NotImplementedError("write your pallas kernel here")


# speedup vs baseline: 17.9998x; 17.9998x over previous
import functools

import jax
import jax.numpy as jnp
from jax import lax
from jax.experimental import pallas as pl
from jax.experimental.pallas import tpu as pltpu
from jax.experimental.pallas import tpu_sc as plsc

N = 100000
EMB = 16
E = 3200000
NUMCLS = 40

NC = 2
NS = 16
NW = NC * NS
IDXW = 128
K = 8
CHUNK = K * IDXW
EPW_GRAN = NW * CHUNK
EP = ((E + EPW_GRAN - 1) // EPW_GRAN) * EPW_GRAN
EPW = EP // NW
NCH = EPW // CHUNK

N_ACC = 102400
ZCH = 640
N_CP = 100480
CPD = 1256

NPH = 2
H = 50048
N_OUT = NPH * H
ACC_R = 51200
DUMMY = H

BLK = 5000
GRID = N // BLK

_mesh = plsc.VectorSubcoreMesh(core_axis_name="c", subcore_axis_name="s")
_sc_params = pltpu.CompilerParams(use_tc_tiling_on_sc=False)


def _sc_degree_body(dst_hbm, out_hbm, idx_v, ones_v, zero_v, cp_v, acc):
    c = lax.axis_index("c")
    s = lax.axis_index("s")
    wid = s * NC + c
    for i in range(IDXW // 16):
        ones_v[pl.ds(i * 16, 16)] = jnp.ones((16,), jnp.float32)
    for i in range(ZCH // 16):
        zero_v[pl.ds(i * 16, 16)] = jnp.zeros((16,), jnp.float32)

    def zacc(z, carry):
        pltpu.sync_copy(zero_v, acc.at[pl.ds(s * (N_ACC // NS) + z * ZCH, ZCH)])
        return carry

    lax.fori_loop(0, N_ACC // NS // ZCH, zacc, 0)
    plsc.subcore_barrier()

    def chunk(ch, carry):
        rbase = wid * (EPW // IDXW) + ch * K
        pltpu.sync_copy(dst_hbm.at[pl.ds(rbase, K)], idx_v)
        for j in range(K):
            pltpu.sync_copy(ones_v, acc.at[idx_v.at[j]], add=True)
        return carry

    lax.fori_loop(0, NCH, chunk, 0)
    plsc.subcore_barrier()

    def cp(z, carry):
        off = s * (N_CP // NS) + z * CPD
        pltpu.sync_copy(acc.at[pl.ds(off, CPD)], cp_v)
        pltpu.sync_copy(cp_v, out_hbm.at[pl.ds(c * N_CP + off, CPD)])
        return carry

    lax.fori_loop(0, N_CP // NS // CPD, cp, 0)


def _sc_degree(dst2d):
    return pl.kernel(
        _sc_degree_body,
        out_type=jax.ShapeDtypeStruct((NC * N_CP,), jnp.float32),
        mesh=_mesh,
        compiler_params=_sc_params,
        scratch_types=[
            pltpu.VMEM((K, IDXW), jnp.int32),
            pltpu.VMEM((IDXW,), jnp.float32),
            pltpu.VMEM((ZCH,), jnp.float32),
            pltpu.VMEM((CPD,), jnp.float32),
            pltpu.VMEM_SHARED((N_ACC,), jnp.float32),
        ],
    )(dst2d)


def _sc_scatter_body(y_hbm, src_hbm, dst_hbm, out_hbm,
                     sidx, didx, didx2, rows, zrows, cbuf, acc, sem):
    c = lax.axis_index("c")
    s = lax.axis_index("s")
    wid = s * NC + c

    def zinit(i, carry):
        zrows[i] = jnp.zeros((16,), jnp.float32)
        return carry

    lax.fori_loop(0, ZCH, zinit, 0)

    for p in range(NPH):
        lo = p * H

        def zacc(z, carry):
            pltpu.sync_copy(zrows, acc.at[pl.ds(s * (ACC_R // NS) + z * ZCH, ZCH)])
            return carry

        lax.fori_loop(0, ACC_R // NS // ZCH, zacc, 0)
        plsc.subcore_barrier()

        def chunk(ch, carry):
            rbase = wid * (EPW // IDXW) + ch * K
            pltpu.sync_copy(src_hbm.at[pl.ds(rbase, K)], sidx)
            pltpu.sync_copy(dst_hbm.at[pl.ds(rbase, K)], didx)
            for j in range(K):
                for l in range(IDXW // 16):
                    v = didx[j, pl.ds(l * 16, 16)]
                    t = v - lo
                    m = (t >= 0) & (t < H)
                    didx2[j, pl.ds(l * 16, 16)] = jnp.where(m, t, DUMMY)
            cps = [
                pltpu.async_copy(y_hbm.at[sidx.at[j]],
                                 rows.at[pl.ds(j * IDXW, IDXW)], sem)
                for j in range(K)
            ]
            for cp in cps:
                cp.wait()
            for j in range(K):
                pltpu.sync_copy(rows.at[pl.ds(j * IDXW, IDXW)],
                                acc.at[didx2.at[j]], add=True)
            return carry

        lax.fori_loop(0, NCH, chunk, 0)
        plsc.subcore_barrier()

        off = s * (H // NS)
        pltpu.sync_copy(acc.at[pl.ds(off, H // NS)], cbuf)
        pltpu.sync_copy(cbuf, out_hbm.at[c, pl.ds(lo + off, H // NS)])
        plsc.subcore_barrier()


_sc_scatter = pl.kernel(
    _sc_scatter_body,
    out_type=jax.ShapeDtypeStruct((NC, N_OUT, EMB), jnp.float32),
    mesh=_mesh,
    compiler_params=_sc_params,
    scratch_types=[
        pltpu.VMEM((K, IDXW), jnp.int32),
        pltpu.VMEM((K, IDXW), jnp.int32),
        pltpu.VMEM((K, IDXW), jnp.int32),
        pltpu.VMEM((CHUNK, EMB), jnp.float32),
        pltpu.VMEM((ZCH, EMB), jnp.float32),
        pltpu.VMEM((H // NS, EMB), jnp.float32),
        pltpu.VMEM_SHARED((ACC_R, EMB), jnp.float32),
        pltpu.SemaphoreType.DMA,
    ],
)


def _prep_body(deg_ref, x_ref, dinv_ref, y_ref):
    d = jnp.sum(deg_ref[...], axis=1, keepdims=True) + 1.0
    dinv = lax.rsqrt(d)
    dinv16 = jnp.broadcast_to(dinv, (BLK, EMB))
    dinv_ref[...] = dinv16
    y_ref[...] = x_ref[...] * dinv16


def _tc_prep(degT, x):
    return pl.pallas_call(
        _prep_body,
        grid=(GRID,),
        in_specs=[
            pl.BlockSpec((BLK, NC), lambda i: (i, 0)),
            pl.BlockSpec((BLK, EMB), lambda i: (i, 0)),
        ],
        out_specs=[
            pl.BlockSpec((BLK, EMB), lambda i: (i, 0)),
            pl.BlockSpec((BLK, EMB), lambda i: (i, 0)),
        ],
        out_shape=[
            jax.ShapeDtypeStruct((N, EMB), jnp.float32),
            jax.ShapeDtypeStruct((N, EMB), jnp.float32),
        ],
    )(degT, x)


def _accum_stats(st_ref, t, i):
    s1 = jnp.sum(t, axis=0, keepdims=True)
    s2 = jnp.sum(t * t, axis=0, keepdims=True)
    blk = jnp.concatenate([s1, s2, jnp.zeros((6, t.shape[1]), jnp.float32)],
                          axis=0)

    @pl.when(i == 0)
    def _():
        st_ref[...] = blk

    @pl.when(i > 0)
    def _():
        st_ref[...] = st_ref[...] + blk


def _mix_body(p_ref, x_ref, di_ref, wg_ref, bg_ref, t_ref, st_ref):
    i = pl.program_id(0)
    s = p_ref[0] + p_ref[1]
    di = di_ref[...]
    x = x_ref[...]
    agg = di * s + di * di * x
    t = jnp.dot(agg, wg_ref[...], preferred_element_type=jnp.float32)
    t = t + bg_ref[...] + x
    t_ref[...] = t
    _accum_stats(st_ref, t, i)


def _tc_mix(part, x, dinv16, wg, bg):
    return pl.pallas_call(
        _mix_body,
        grid=(GRID,),
        in_specs=[
            pl.BlockSpec((NC, BLK, EMB), lambda i: (0, i, 0)),
            pl.BlockSpec((BLK, EMB), lambda i: (i, 0)),
            pl.BlockSpec((BLK, EMB), lambda i: (i, 0)),
            pl.BlockSpec((EMB, EMB), lambda i: (0, 0)),
            pl.BlockSpec((1, EMB), lambda i: (0, 0)),
        ],
        out_specs=[
            pl.BlockSpec((BLK, EMB), lambda i: (i, 0)),
            pl.BlockSpec((8, EMB), lambda i: (0, 0)),
        ],
        out_shape=[
            jax.ShapeDtypeStruct((N, EMB), jnp.float32),
            jax.ShapeDtypeStruct((8, EMB), jnp.float32),
        ],
    )(part, x, dinv16, wg, bg)


def _bn(t, st, g, b):
    mu = st[0:1] * (1.0 / N)
    var = st[1:2] * (1.0 / N) - mu * mu
    rstd = lax.rsqrt(var + 1e-5)
    return (t - mu) * rstd * g + b


def _ff_body(t_ref, st_ref, g_ref, be_ref, w1_ref, bb1_ref, w2_ref,
             u_ref, st2_ref):
    i = pl.program_id(0)
    h = _bn(t_ref[...], st_ref[...], g_ref[...], be_ref[...])
    a = jnp.dot(h, w1_ref[...], preferred_element_type=jnp.float32)
    a = jnp.maximum(a + bb1_ref[...], 0.0)
    u = jnp.dot(a, w2_ref[...], preferred_element_type=jnp.float32) + h
    u_ref[...] = u
    _accum_stats(st2_ref, u, i)


def _tc_ff(t, st1, g1, be1, w1, bb1, w2):
    M = w1.shape[1]
    return pl.pallas_call(
        _ff_body,
        grid=(GRID,),
        in_specs=[
            pl.BlockSpec((BLK, EMB), lambda i: (i, 0)),
            pl.BlockSpec((8, EMB), lambda i: (0, 0)),
            pl.BlockSpec((1, EMB), lambda i: (0, 0)),
            pl.BlockSpec((1, EMB), lambda i: (0, 0)),
            pl.BlockSpec((EMB, M), lambda i: (0, 0)),
            pl.BlockSpec((1, M), lambda i: (0, 0)),
            pl.BlockSpec((M, EMB), lambda i: (0, 0)),
        ],
        out_specs=[
            pl.BlockSpec((BLK, EMB), lambda i: (i, 0)),
            pl.BlockSpec((8, EMB), lambda i: (0, 0)),
        ],
        out_shape=[
            jax.ShapeDtypeStruct((N, EMB), jnp.float32),
            jax.ShapeDtypeStruct((8, EMB), jnp.float32),
        ],
    )(t, st1, g1, be1, w1, bb1, w2)


def _bnout_body(u_ref, st_ref, g_ref, be_ref, di_ref, xn_ref, y_ref):
    xn = _bn(u_ref[...], st_ref[...], g_ref[...], be_ref[...])
    xn_ref[...] = xn
    y_ref[...] = di_ref[...] * xn


def _tc_bnout(u, st2, g2, be2, dinv16):
    return pl.pallas_call(
        _bnout_body,
        grid=(GRID,),
        in_specs=[
            pl.BlockSpec((BLK, EMB), lambda i: (i, 0)),
            pl.BlockSpec((8, EMB), lambda i: (0, 0)),
            pl.BlockSpec((1, EMB), lambda i: (0, 0)),
            pl.BlockSpec((1, EMB), lambda i: (0, 0)),
            pl.BlockSpec((BLK, EMB), lambda i: (i, 0)),
        ],
        out_specs=[
            pl.BlockSpec((BLK, EMB), lambda i: (i, 0)),
            pl.BlockSpec((BLK, EMB), lambda i: (i, 0)),
        ],
        out_shape=[
            jax.ShapeDtypeStruct((N, EMB), jnp.float32),
            jax.ShapeDtypeStruct((N, EMB), jnp.float32),
        ],
    )(u, st2, g2, be2, dinv16)


def _cls_body(u_ref, st_ref, g_ref, be_ref, w_ref, b_ref, o_ref):
    xn = _bn(u_ref[...], st_ref[...], g_ref[...], be_ref[...])
    o = jnp.dot(xn, w_ref[...], preferred_element_type=jnp.float32)
    o_ref[...] = o + b_ref[...]


def _tc_cls(u, st2, g2, be2, w, b):
    return pl.pallas_call(
        _cls_body,
        grid=(GRID,),
        in_specs=[
            pl.BlockSpec((BLK, EMB), lambda i: (i, 0)),
            pl.BlockSpec((8, EMB), lambda i: (0, 0)),
            pl.BlockSpec((1, EMB), lambda i: (0, 0)),
            pl.BlockSpec((1, EMB), lambda i: (0, 0)),
            pl.BlockSpec((EMB, NUMCLS), lambda i: (0, 0)),
            pl.BlockSpec((1, NUMCLS), lambda i: (0, 0)),
        ],
        out_specs=pl.BlockSpec((BLK, NUMCLS), lambda i: (i, 0)),
        out_shape=jax.ShapeDtypeStruct((N, NUMCLS), jnp.float32),
    )(u, st2, g2, be2, w, b)


def kernel(params, edge_index):
    pad = EP - E
    srcp = jnp.concatenate(
        [edge_index[0], jnp.zeros((pad,), jnp.int32)]).reshape(EP // IDXW, IDXW)
    dstp = jnp.concatenate(
        [edge_index[1], jnp.full((pad,), N, jnp.int32)]).reshape(EP // IDXW, IDXW)

    degp = _sc_degree(dstp).reshape(NC, N_CP)
    degT = jnp.transpose(degp[:, :N])
    x = params["nodes"]
    dinv16, y = _tc_prep(degT, x)

    out = None
    for i in range(2):
        part = _sc_scatter(y, srcp, dstp)[:, :N]
        t, st1 = _tc_mix(part, x, dinv16,
                         params[f"b{i}_wg"], params[f"b{i}_bg"].reshape(1, EMB))
        u, st2 = _tc_ff(t, st1,
                        params[f"b{i}_g1"].reshape(1, EMB),
                        params[f"b{i}_be1"].reshape(1, EMB),
                        params[f"b{i}_w1"],
                        params[f"b{i}_bb1"].reshape(1, 4 * EMB),
                        params[f"b{i}_w2"])
        if i == 0:
            x, y = _tc_bnout(u, st2,
                             params["b0_g2"].reshape(1, EMB),
                             params["b0_be2"].reshape(1, EMB), dinv16)
        else:
            out = _tc_cls(u, st2,
                          params["b1_g2"].reshape(1, EMB),
                          params["b1_be2"].reshape(1, EMB),
                          params["cls_w"], params["cls_b"].reshape(1, NUMCLS))
    return out


# speedup vs baseline: 32.4836x; 1.8047x over previous
import functools

import jax
import jax.numpy as jnp
from jax import lax
from jax.experimental import pallas as pl
from jax.experimental.pallas import tpu as pltpu
from jax.experimental.pallas import tpu_sc as plsc

N = 100000
EMB = 16
E = 3200000
NUMCLS = 40

NC = 2
NS = 16
NW = NC * NS
IDXW = 128
K = 16
CHUNK = K * IDXW
EPW_GRAN = NW * CHUNK
EP = ((E + EPW_GRAN - 1) // EPW_GRAN) * EPW_GRAN
EPW = EP // NW
NCH = EPW // CHUNK

N_ACC = 102400
DZ = 640
ZCH = 560
N_CP = 100480
CPD = 1256

NPH = 3
H = 33408
N_OUT = NPH * H
ACC_R = 35840
DUMMY = H

BLK = 5000
GRID = N // BLK

_mesh = plsc.VectorSubcoreMesh(core_axis_name="c", subcore_axis_name="s")
_sc_params = pltpu.CompilerParams(use_tc_tiling_on_sc=False)


def _sc_degree_body(dst_hbm, out_hbm, idx_v, ones_v, zero_v, cp_v, acc):
    c = lax.axis_index("c")
    s = lax.axis_index("s")
    wid = s * NC + c
    for i in range(IDXW // 16):
        ones_v[pl.ds(i * 16, 16)] = jnp.ones((16,), jnp.float32)
    for i in range(DZ // 16):
        zero_v[pl.ds(i * 16, 16)] = jnp.zeros((16,), jnp.float32)

    def zacc(z, carry):
        pltpu.sync_copy(zero_v, acc.at[pl.ds(s * (N_ACC // NS) + z * DZ, DZ)])
        return carry

    lax.fori_loop(0, N_ACC // NS // DZ, zacc, 0)
    plsc.subcore_barrier()

    def chunk(ch, carry):
        rbase = wid * (EPW // IDXW) + ch * K
        pltpu.sync_copy(dst_hbm.at[pl.ds(rbase, K)], idx_v)
        for j in range(K):
            pltpu.sync_copy(ones_v, acc.at[idx_v.at[j]], add=True)
        return carry

    lax.fori_loop(0, NCH, chunk, 0)
    plsc.subcore_barrier()

    def cp(z, carry):
        off = s * (N_CP // NS) + z * CPD
        pltpu.sync_copy(acc.at[pl.ds(off, CPD)], cp_v)
        pltpu.sync_copy(cp_v, out_hbm.at[pl.ds(c * N_CP + off, CPD)])
        return carry

    lax.fori_loop(0, N_CP // NS // CPD, cp, 0)


def _sc_degree(dst2d):
    return pl.kernel(
        _sc_degree_body,
        out_type=jax.ShapeDtypeStruct((NC * N_CP,), jnp.float32),
        mesh=_mesh,
        compiler_params=_sc_params,
        scratch_types=[
            pltpu.VMEM((K, IDXW), jnp.int32),
            pltpu.VMEM((IDXW,), jnp.float32),
            pltpu.VMEM((DZ,), jnp.float32),
            pltpu.VMEM((CPD,), jnp.float32),
            pltpu.VMEM_SHARED((N_ACC,), jnp.float32),
        ],
    )(dst2d)


def _sc_scatter_body(y_hbm, src_hbm, dst_hbm, out_hbm,
                     sidx, didx, didx2, rows, zrows, cbuf, acc, sem, sem2):
    c = lax.axis_index("c")
    s = lax.axis_index("s")
    wid = s * NC + c

    def zinit(i, carry):
        zrows[i] = jnp.zeros((16,), jnp.float32)
        return carry

    lax.fori_loop(0, ZCH, zinit, 0)

    for p in range(NPH):
        lo = p * H

        def zacc(z, carry):
            pltpu.sync_copy(zrows, acc.at[pl.ds(s * (ACC_R // NS) + z * ZCH, ZCH)])
            return carry

        lax.fori_loop(0, ACC_R // NS // ZCH, zacc, 0)
        plsc.subcore_barrier()

        def chunk(ch, carry):
            rbase = wid * (EPW // IDXW) + ch * K
            pltpu.sync_copy(src_hbm.at[pl.ds(rbase, K)], sidx)
            pltpu.sync_copy(dst_hbm.at[pl.ds(rbase, K)], didx)
            gs = [
                pltpu.async_copy(y_hbm.at[sidx.at[j]],
                                 rows.at[pl.ds(j * IDXW, IDXW)], sem)
                for j in range(K)
            ]
            for j in range(K):
                for l in range(IDXW // 16):
                    v = didx[j, pl.ds(l * 16, 16)]
                    t = v - lo
                    m = (t >= 0) & (t < H)
                    didx2[j, pl.ds(l * 16, 16)] = jnp.where(
                        m, t, DUMMY + (v & 1023))
            ss = []
            for j in range(K):
                gs[j].wait()
                ss.append(pltpu.async_copy(rows.at[pl.ds(j * IDXW, IDXW)],
                                           acc.at[didx2.at[j]], sem2,
                                           add=True))
            for cp in ss:
                cp.wait()
            return carry

        lax.fori_loop(0, NCH, chunk, 0)
        plsc.subcore_barrier()

        off = s * (H // NS)
        pltpu.sync_copy(acc.at[pl.ds(off, H // NS)], cbuf)
        pltpu.sync_copy(cbuf, out_hbm.at[c, pl.ds(lo + off, H // NS)])
        plsc.subcore_barrier()


_sc_scatter = pl.kernel(
    _sc_scatter_body,
    out_type=jax.ShapeDtypeStruct((NC, N_OUT, EMB), jnp.float32),
    mesh=_mesh,
    compiler_params=_sc_params,
    scratch_types=[
        pltpu.VMEM((K, IDXW), jnp.int32),
        pltpu.VMEM((K, IDXW), jnp.int32),
        pltpu.VMEM((K, IDXW), jnp.int32),
        pltpu.VMEM((CHUNK, EMB), jnp.float32),
        pltpu.VMEM((ZCH, EMB), jnp.float32),
        pltpu.VMEM((H // NS, EMB), jnp.float32),
        pltpu.VMEM_SHARED((ACC_R, EMB), jnp.float32),
        pltpu.SemaphoreType.DMA,
        pltpu.SemaphoreType.DMA,
    ],
)


def _prep_body(deg_ref, x_ref, dinv_ref, y_ref):
    d = jnp.sum(deg_ref[...], axis=1, keepdims=True) + 1.0
    dinv = lax.rsqrt(d)
    dinv16 = jnp.broadcast_to(dinv, (BLK, EMB))
    dinv_ref[...] = dinv16
    y_ref[...] = x_ref[...] * dinv16


def _tc_prep(degT, x):
    return pl.pallas_call(
        _prep_body,
        grid=(GRID,),
        in_specs=[
            pl.BlockSpec((BLK, NC), lambda i: (i, 0)),
            pl.BlockSpec((BLK, EMB), lambda i: (i, 0)),
        ],
        out_specs=[
            pl.BlockSpec((BLK, EMB), lambda i: (i, 0)),
            pl.BlockSpec((BLK, EMB), lambda i: (i, 0)),
        ],
        out_shape=[
            jax.ShapeDtypeStruct((N, EMB), jnp.float32),
            jax.ShapeDtypeStruct((N, EMB), jnp.float32),
        ],
    )(degT, x)


def _accum_stats(st_ref, t, i):
    s1 = jnp.sum(t, axis=0, keepdims=True)
    s2 = jnp.sum(t * t, axis=0, keepdims=True)
    blk = jnp.concatenate([s1, s2, jnp.zeros((6, t.shape[1]), jnp.float32)],
                          axis=0)

    @pl.when(i == 0)
    def _():
        st_ref[...] = blk

    @pl.when(i > 0)
    def _():
        st_ref[...] = st_ref[...] + blk


def _mix_body(p_ref, x_ref, di_ref, wg_ref, bg_ref, t_ref, st_ref):
    i = pl.program_id(0)
    s = p_ref[0] + p_ref[1]
    di = di_ref[...]
    x = x_ref[...]
    agg = di * s + di * di * x
    t = jnp.dot(agg, wg_ref[...], preferred_element_type=jnp.float32)
    t = t + bg_ref[...] + x
    t_ref[...] = t
    _accum_stats(st_ref, t, i)


def _tc_mix(part, x, dinv16, wg, bg):
    return pl.pallas_call(
        _mix_body,
        grid=(GRID,),
        in_specs=[
            pl.BlockSpec((NC, BLK, EMB), lambda i: (0, i, 0)),
            pl.BlockSpec((BLK, EMB), lambda i: (i, 0)),
            pl.BlockSpec((BLK, EMB), lambda i: (i, 0)),
            pl.BlockSpec((EMB, EMB), lambda i: (0, 0)),
            pl.BlockSpec((1, EMB), lambda i: (0, 0)),
        ],
        out_specs=[
            pl.BlockSpec((BLK, EMB), lambda i: (i, 0)),
            pl.BlockSpec((8, EMB), lambda i: (0, 0)),
        ],
        out_shape=[
            jax.ShapeDtypeStruct((N, EMB), jnp.float32),
            jax.ShapeDtypeStruct((8, EMB), jnp.float32),
        ],
    )(part, x, dinv16, wg, bg)


def _bn(t, st, g, b):
    mu = st[0:1] * (1.0 / N)
    var = st[1:2] * (1.0 / N) - mu * mu
    rstd = lax.rsqrt(var + 1e-5)
    return (t - mu) * rstd * g + b


def _ff_body(t_ref, st_ref, g_ref, be_ref, w1_ref, bb1_ref, w2_ref,
             u_ref, st2_ref):
    i = pl.program_id(0)
    h = _bn(t_ref[...], st_ref[...], g_ref[...], be_ref[...])
    a = jnp.dot(h, w1_ref[...], preferred_element_type=jnp.float32)
    a = jnp.maximum(a + bb1_ref[...], 0.0)
    u = jnp.dot(a, w2_ref[...], preferred_element_type=jnp.float32) + h
    u_ref[...] = u
    _accum_stats(st2_ref, u, i)


def _tc_ff(t, st1, g1, be1, w1, bb1, w2):
    M = w1.shape[1]
    return pl.pallas_call(
        _ff_body,
        grid=(GRID,),
        in_specs=[
            pl.BlockSpec((BLK, EMB), lambda i: (i, 0)),
            pl.BlockSpec((8, EMB), lambda i: (0, 0)),
            pl.BlockSpec((1, EMB), lambda i: (0, 0)),
            pl.BlockSpec((1, EMB), lambda i: (0, 0)),
            pl.BlockSpec((EMB, M), lambda i: (0, 0)),
            pl.BlockSpec((1, M), lambda i: (0, 0)),
            pl.BlockSpec((M, EMB), lambda i: (0, 0)),
        ],
        out_specs=[
            pl.BlockSpec((BLK, EMB), lambda i: (i, 0)),
            pl.BlockSpec((8, EMB), lambda i: (0, 0)),
        ],
        out_shape=[
            jax.ShapeDtypeStruct((N, EMB), jnp.float32),
            jax.ShapeDtypeStruct((8, EMB), jnp.float32),
        ],
    )(t, st1, g1, be1, w1, bb1, w2)


def _bnout_body(u_ref, st_ref, g_ref, be_ref, di_ref, xn_ref, y_ref):
    xn = _bn(u_ref[...], st_ref[...], g_ref[...], be_ref[...])
    xn_ref[...] = xn
    y_ref[...] = di_ref[...] * xn


def _tc_bnout(u, st2, g2, be2, dinv16):
    return pl.pallas_call(
        _bnout_body,
        grid=(GRID,),
        in_specs=[
            pl.BlockSpec((BLK, EMB), lambda i: (i, 0)),
            pl.BlockSpec((8, EMB), lambda i: (0, 0)),
            pl.BlockSpec((1, EMB), lambda i: (0, 0)),
            pl.BlockSpec((1, EMB), lambda i: (0, 0)),
            pl.BlockSpec((BLK, EMB), lambda i: (i, 0)),
        ],
        out_specs=[
            pl.BlockSpec((BLK, EMB), lambda i: (i, 0)),
            pl.BlockSpec((BLK, EMB), lambda i: (i, 0)),
        ],
        out_shape=[
            jax.ShapeDtypeStruct((N, EMB), jnp.float32),
            jax.ShapeDtypeStruct((N, EMB), jnp.float32),
        ],
    )(u, st2, g2, be2, dinv16)


def _cls_body(x_ref, w_ref, b_ref, o_ref):
    o = jnp.dot(x_ref[...], w_ref[...], preferred_element_type=jnp.float32)
    o_ref[...] = o + b_ref[...]


def _tc_cls(x, w, b):
    return pl.pallas_call(
        _cls_body,
        grid=(GRID,),
        in_specs=[
            pl.BlockSpec((BLK, EMB), lambda i: (i, 0)),
            pl.BlockSpec((EMB, NUMCLS), lambda i: (0, 0)),
            pl.BlockSpec((1, NUMCLS), lambda i: (0, 0)),
        ],
        out_specs=pl.BlockSpec((BLK, NUMCLS), lambda i: (i, 0)),
        out_shape=jax.ShapeDtypeStruct((N, NUMCLS), jnp.float32),
    )(x, w, b)


def kernel(params, edge_index):
    pad = EP - E
    srcp = jnp.concatenate(
        [edge_index[0], jnp.zeros((pad,), jnp.int32)]).reshape(EP // IDXW, IDXW)
    dstp = jnp.concatenate(
        [edge_index[1], jnp.full((pad,), N, jnp.int32)]).reshape(EP // IDXW, IDXW)

    degp = _sc_degree(dstp).reshape(NC, N_CP)
    degT = jnp.transpose(degp[:, :N])
    x = params["nodes"]
    dinv16, y = _tc_prep(degT, x)

    stk = {
        "wg": jnp.stack([params["b0_wg"], params["b1_wg"]]),
        "bg": jnp.stack([params["b0_bg"].reshape(1, EMB),
                         params["b1_bg"].reshape(1, EMB)]),
        "g1": jnp.stack([params["b0_g1"].reshape(1, EMB),
                         params["b1_g1"].reshape(1, EMB)]),
        "be1": jnp.stack([params["b0_be1"].reshape(1, EMB),
                          params["b1_be1"].reshape(1, EMB)]),
        "w1": jnp.stack([params["b0_w1"], params["b1_w1"]]),
        "bb1": jnp.stack([params["b0_bb1"].reshape(1, 4 * EMB),
                          params["b1_bb1"].reshape(1, 4 * EMB)]),
        "w2": jnp.stack([params["b0_w2"], params["b1_w2"]]),
        "g2": jnp.stack([params["b0_g2"].reshape(1, EMB),
                         params["b1_g2"].reshape(1, EMB)]),
        "be2": jnp.stack([params["b0_be2"].reshape(1, EMB),
                          params["b1_be2"].reshape(1, EMB)]),
    }

    def step(carry, w):
        xc, yc = carry
        part = _sc_scatter(yc, srcp, dstp)[:, :N]
        t, st1 = _tc_mix(part, xc, dinv16, w["wg"], w["bg"])
        u, st2 = _tc_ff(t, st1, w["g1"], w["be1"], w["w1"], w["bb1"], w["w2"])
        xn, yn = _tc_bnout(u, st2, w["g2"], w["be2"], dinv16)
        return (xn, yn), None

    (x, y), _ = lax.scan(step, (x, y), stk)
    return _tc_cls(x, params["cls_w"], params["cls_b"].reshape(1, NUMCLS))


# speedup vs baseline: 35.4907x; 1.0926x over previous
import functools

import jax
import jax.numpy as jnp
from jax import lax
from jax.experimental import pallas as pl
from jax.experimental.pallas import tpu as pltpu
from jax.experimental.pallas import tpu_sc as plsc

N = 100000
EMB = 16
E = 3200000
NUMCLS = 40

NC = 2
NS = 16
NW = NC * NS
IDXW = 128
K = 8
CHUNK = K * IDXW
EPW_GRAN = NW * CHUNK
EP = ((E + EPW_GRAN - 1) // EPW_GRAN) * EPW_GRAN
EPW = EP // NW
NCH = EPW // CHUNK

N_ACC = 102400
DZ = 640
ZCH = 400
N_CP = 100480
CPD = 1256

NPH = 2
H = 50048
N_OUT = NPH * H
ACC_R = 51200
DUMMY = H

BLK = 5000
GRID = N // BLK

_mesh = plsc.VectorSubcoreMesh(core_axis_name="c", subcore_axis_name="s")
_sc_params = pltpu.CompilerParams(use_tc_tiling_on_sc=False)


def _sc_degree_body(dst_hbm, out_hbm, idx_v, ones_v, zero_v, cp_v, acc):
    c = lax.axis_index("c")
    s = lax.axis_index("s")
    wid = s * NC + c
    for i in range(IDXW // 16):
        ones_v[pl.ds(i * 16, 16)] = jnp.ones((16,), jnp.float32)
    for i in range(DZ // 16):
        zero_v[pl.ds(i * 16, 16)] = jnp.zeros((16,), jnp.float32)

    def zacc(z, carry):
        pltpu.sync_copy(zero_v, acc.at[pl.ds(s * (N_ACC // NS) + z * DZ, DZ)])
        return carry

    lax.fori_loop(0, N_ACC // NS // DZ, zacc, 0)
    plsc.subcore_barrier()

    def chunk(ch, carry):
        rbase = wid * (EPW // IDXW) + ch * K
        pltpu.sync_copy(dst_hbm.at[pl.ds(rbase, K)], idx_v)
        for j in range(K):
            pltpu.sync_copy(ones_v, acc.at[idx_v.at[j]], add=True)
        return carry

    lax.fori_loop(0, NCH, chunk, 0)
    plsc.subcore_barrier()

    def cp(z, carry):
        off = s * (N_CP // NS) + z * CPD
        pltpu.sync_copy(acc.at[pl.ds(off, CPD)], cp_v)
        pltpu.sync_copy(cp_v, out_hbm.at[pl.ds(c * N_CP + off, CPD)])
        return carry

    lax.fori_loop(0, N_CP // NS // CPD, cp, 0)


def _sc_degree(dst2d):
    return pl.kernel(
        _sc_degree_body,
        out_type=jax.ShapeDtypeStruct((NC * N_CP,), jnp.float32),
        mesh=_mesh,
        compiler_params=_sc_params,
        scratch_types=[
            pltpu.VMEM((K, IDXW), jnp.int32),
            pltpu.VMEM((IDXW,), jnp.float32),
            pltpu.VMEM((DZ,), jnp.float32),
            pltpu.VMEM((CPD,), jnp.float32),
            pltpu.VMEM_SHARED((N_ACC,), jnp.float32),
        ],
    )(dst2d)


def _sc_scatter_body(y_hbm, src_hbm, dst_hbm, out_hbm,
                     sidx, didx, didx2, rows, zrows, cbuf, acc, sem, sem2):
    c = lax.axis_index("c")
    s = lax.axis_index("s")
    wid = s * NC + c

    def zinit(i, carry):
        zrows[i] = jnp.zeros((16,), jnp.float32)
        return carry

    lax.fori_loop(0, ZCH, zinit, 0)

    for p in range(NPH):
        lo = p * H

        def zacc(z, carry):
            pltpu.sync_copy(zrows, acc.at[pl.ds(s * (ACC_R // NS) + z * ZCH, ZCH)])
            return carry

        lax.fori_loop(0, ACC_R // NS // ZCH, zacc, 0)
        plsc.subcore_barrier()

        def chunk(ch, carry):
            rbase = wid * (EPW // IDXW) + ch * K
            pltpu.sync_copy(src_hbm.at[pl.ds(rbase, K)], sidx)
            pltpu.sync_copy(dst_hbm.at[pl.ds(rbase, K)], didx)
            gs = [
                pltpu.async_copy(y_hbm.at[sidx.at[j]],
                                 rows.at[pl.ds(j * IDXW, IDXW)], sem)
                for j in range(K)
            ]
            for j in range(K):
                for l in range(IDXW // 16):
                    v = didx[j, pl.ds(l * 16, 16)]
                    t = v - lo
                    m = (t >= 0) & (t < H)
                    didx2[j, pl.ds(l * 16, 16)] = jnp.where(
                        m, t, DUMMY + (v & 1023))
            ss = []
            for j in range(K):
                gs[j].wait()
                ss.append(pltpu.async_copy(rows.at[pl.ds(j * IDXW, IDXW)],
                                           acc.at[didx2.at[j]], sem2,
                                           add=True))
            for cp in ss:
                cp.wait()
            return carry

        lax.fori_loop(0, NCH, chunk, 0)
        plsc.subcore_barrier()

        off = s * (H // NS)
        pltpu.sync_copy(acc.at[pl.ds(off, H // NS)], cbuf)
        pltpu.sync_copy(cbuf, out_hbm.at[c, pl.ds(lo + off, H // NS)])
        plsc.subcore_barrier()


_sc_scatter = pl.kernel(
    _sc_scatter_body,
    out_type=jax.ShapeDtypeStruct((NC, N_OUT, EMB), jnp.float32),
    mesh=_mesh,
    compiler_params=_sc_params,
    scratch_types=[
        pltpu.VMEM((K, IDXW), jnp.int32),
        pltpu.VMEM((K, IDXW), jnp.int32),
        pltpu.VMEM((K, IDXW), jnp.int32),
        pltpu.VMEM((CHUNK, EMB), jnp.float32),
        pltpu.VMEM((ZCH, EMB), jnp.float32),
        pltpu.VMEM((H // NS, EMB), jnp.float32),
        pltpu.VMEM_SHARED((ACC_R, EMB), jnp.float32),
        pltpu.SemaphoreType.DMA,
        pltpu.SemaphoreType.DMA,
    ],
)


def _prep_body(deg_ref, x_ref, dinv_ref, y_ref):
    d = jnp.sum(deg_ref[...], axis=1, keepdims=True) + 1.0
    dinv = lax.rsqrt(d)
    dinv16 = jnp.broadcast_to(dinv, (BLK, EMB))
    dinv_ref[...] = dinv16
    y_ref[...] = x_ref[...] * dinv16


def _tc_prep(degT, x):
    return pl.pallas_call(
        _prep_body,
        grid=(GRID,),
        in_specs=[
            pl.BlockSpec((BLK, NC), lambda i: (i, 0)),
            pl.BlockSpec((BLK, EMB), lambda i: (i, 0)),
        ],
        out_specs=[
            pl.BlockSpec((BLK, EMB), lambda i: (i, 0)),
            pl.BlockSpec((BLK, EMB), lambda i: (i, 0)),
        ],
        out_shape=[
            jax.ShapeDtypeStruct((N, EMB), jnp.float32),
            jax.ShapeDtypeStruct((N, EMB), jnp.float32),
        ],
    )(degT, x)


def _accum_stats(st_ref, t, i):
    s1 = jnp.sum(t, axis=0, keepdims=True)
    s2 = jnp.sum(t * t, axis=0, keepdims=True)
    blk = jnp.concatenate([s1, s2, jnp.zeros((6, t.shape[1]), jnp.float32)],
                          axis=0)

    @pl.when(i == 0)
    def _():
        st_ref[...] = blk

    @pl.when(i > 0)
    def _():
        st_ref[...] = st_ref[...] + blk


def _mix_body(p_ref, x_ref, di_ref, wg_ref, bg_ref, t_ref, st_ref):
    i = pl.program_id(0)
    s = p_ref[0] + p_ref[1]
    di = di_ref[...]
    x = x_ref[...]
    agg = di * s + di * di * x
    t = jnp.dot(agg, wg_ref[...], preferred_element_type=jnp.float32)
    t = t + bg_ref[...] + x
    t_ref[...] = t
    _accum_stats(st_ref, t, i)


def _tc_mix(part, x, dinv16, wg, bg):
    return pl.pallas_call(
        _mix_body,
        grid=(GRID,),
        in_specs=[
            pl.BlockSpec((NC, BLK, EMB), lambda i: (0, i, 0)),
            pl.BlockSpec((BLK, EMB), lambda i: (i, 0)),
            pl.BlockSpec((BLK, EMB), lambda i: (i, 0)),
            pl.BlockSpec((EMB, EMB), lambda i: (0, 0)),
            pl.BlockSpec((1, EMB), lambda i: (0, 0)),
        ],
        out_specs=[
            pl.BlockSpec((BLK, EMB), lambda i: (i, 0)),
            pl.BlockSpec((8, EMB), lambda i: (0, 0)),
        ],
        out_shape=[
            jax.ShapeDtypeStruct((N, EMB), jnp.float32),
            jax.ShapeDtypeStruct((8, EMB), jnp.float32),
        ],
    )(part, x, dinv16, wg, bg)


def _bn(t, st, g, b):
    mu = st[0:1] * (1.0 / N)
    var = st[1:2] * (1.0 / N) - mu * mu
    rstd = lax.rsqrt(var + 1e-5)
    return (t - mu) * rstd * g + b


def _ff_body(t_ref, st_ref, g_ref, be_ref, w1_ref, bb1_ref, w2_ref,
             u_ref, st2_ref):
    i = pl.program_id(0)
    h = _bn(t_ref[...], st_ref[...], g_ref[...], be_ref[...])
    a = jnp.dot(h, w1_ref[...], preferred_element_type=jnp.float32)
    a = jnp.maximum(a + bb1_ref[...], 0.0)
    u = jnp.dot(a, w2_ref[...], preferred_element_type=jnp.float32) + h
    u_ref[...] = u
    _accum_stats(st2_ref, u, i)


def _tc_ff(t, st1, g1, be1, w1, bb1, w2):
    M = w1.shape[1]
    return pl.pallas_call(
        _ff_body,
        grid=(GRID,),
        in_specs=[
            pl.BlockSpec((BLK, EMB), lambda i: (i, 0)),
            pl.BlockSpec((8, EMB), lambda i: (0, 0)),
            pl.BlockSpec((1, EMB), lambda i: (0, 0)),
            pl.BlockSpec((1, EMB), lambda i: (0, 0)),
            pl.BlockSpec((EMB, M), lambda i: (0, 0)),
            pl.BlockSpec((1, M), lambda i: (0, 0)),
            pl.BlockSpec((M, EMB), lambda i: (0, 0)),
        ],
        out_specs=[
            pl.BlockSpec((BLK, EMB), lambda i: (i, 0)),
            pl.BlockSpec((8, EMB), lambda i: (0, 0)),
        ],
        out_shape=[
            jax.ShapeDtypeStruct((N, EMB), jnp.float32),
            jax.ShapeDtypeStruct((8, EMB), jnp.float32),
        ],
    )(t, st1, g1, be1, w1, bb1, w2)


def _bnout_body(u_ref, st_ref, g_ref, be_ref, di_ref, xn_ref, y_ref):
    xn = _bn(u_ref[...], st_ref[...], g_ref[...], be_ref[...])
    xn_ref[...] = xn
    y_ref[...] = di_ref[...] * xn


def _tc_bnout(u, st2, g2, be2, dinv16):
    return pl.pallas_call(
        _bnout_body,
        grid=(GRID,),
        in_specs=[
            pl.BlockSpec((BLK, EMB), lambda i: (i, 0)),
            pl.BlockSpec((8, EMB), lambda i: (0, 0)),
            pl.BlockSpec((1, EMB), lambda i: (0, 0)),
            pl.BlockSpec((1, EMB), lambda i: (0, 0)),
            pl.BlockSpec((BLK, EMB), lambda i: (i, 0)),
        ],
        out_specs=[
            pl.BlockSpec((BLK, EMB), lambda i: (i, 0)),
            pl.BlockSpec((BLK, EMB), lambda i: (i, 0)),
        ],
        out_shape=[
            jax.ShapeDtypeStruct((N, EMB), jnp.float32),
            jax.ShapeDtypeStruct((N, EMB), jnp.float32),
        ],
    )(u, st2, g2, be2, dinv16)


def _cls_body(x_ref, w_ref, b_ref, o_ref):
    o = jnp.dot(x_ref[...], w_ref[...], preferred_element_type=jnp.float32)
    o_ref[...] = o + b_ref[...]


def _tc_cls(x, w, b):
    return pl.pallas_call(
        _cls_body,
        grid=(GRID,),
        in_specs=[
            pl.BlockSpec((BLK, EMB), lambda i: (i, 0)),
            pl.BlockSpec((EMB, NUMCLS), lambda i: (0, 0)),
            pl.BlockSpec((1, NUMCLS), lambda i: (0, 0)),
        ],
        out_specs=pl.BlockSpec((BLK, NUMCLS), lambda i: (i, 0)),
        out_shape=jax.ShapeDtypeStruct((N, NUMCLS), jnp.float32),
    )(x, w, b)


def kernel(params, edge_index):
    pad = EP - E
    srcp = jnp.concatenate(
        [edge_index[0], jnp.zeros((pad,), jnp.int32)]).reshape(EP // IDXW, IDXW)
    dstp = jnp.concatenate(
        [edge_index[1], jnp.full((pad,), N, jnp.int32)]).reshape(EP // IDXW, IDXW)

    degp = _sc_degree(dstp).reshape(NC, N_CP)
    degT = jnp.transpose(degp[:, :N])
    x = params["nodes"]
    dinv16, y = _tc_prep(degT, x)

    stk = {
        "wg": jnp.stack([params["b0_wg"], params["b1_wg"]]),
        "bg": jnp.stack([params["b0_bg"].reshape(1, EMB),
                         params["b1_bg"].reshape(1, EMB)]),
        "g1": jnp.stack([params["b0_g1"].reshape(1, EMB),
                         params["b1_g1"].reshape(1, EMB)]),
        "be1": jnp.stack([params["b0_be1"].reshape(1, EMB),
                          params["b1_be1"].reshape(1, EMB)]),
        "w1": jnp.stack([params["b0_w1"], params["b1_w1"]]),
        "bb1": jnp.stack([params["b0_bb1"].reshape(1, 4 * EMB),
                          params["b1_bb1"].reshape(1, 4 * EMB)]),
        "w2": jnp.stack([params["b0_w2"], params["b1_w2"]]),
        "g2": jnp.stack([params["b0_g2"].reshape(1, EMB),
                         params["b1_g2"].reshape(1, EMB)]),
        "be2": jnp.stack([params["b0_be2"].reshape(1, EMB),
                          params["b1_be2"].reshape(1, EMB)]),
    }

    def step(carry, w):
        xc, yc = carry
        part = _sc_scatter(yc, srcp, dstp)[:, :N]
        t, st1 = _tc_mix(part, xc, dinv16, w["wg"], w["bg"])
        u, st2 = _tc_ff(t, st1, w["g1"], w["be1"], w["w1"], w["bb1"], w["w2"])
        xn, yn = _tc_bnout(u, st2, w["g2"], w["be2"], dinv16)
        return (xn, yn), None

    (x, y), _ = lax.scan(step, (x, y), stk)
    return _tc_cls(x, params["cls_w"], params["cls_b"].reshape(1, NUMCLS))
